# baseline (speedup 1.0000x reference)
import functools

import jax
import jax.numpy as jnp
from jax import lax
from jax.experimental import pallas as pl
from jax.experimental.pallas import tpu as pltpu
from jax.experimental.pallas import tpu_sc as plsc

N = 10000
D = 128
E = 320000
NC = 2
NS = 16
NW = NC * NS
EW = E // NW
C = 128
K = 80
KH = K // 2
EW_PAD = K * C
RPW = 632
ACC_ROWS = NS * RPW
TRASH = N
BM = 1000
GRID = N // BM


def _leaky(v):
    return jnp.where(v > 0, v, 0.01 * v)


def _segment_sum_sc(x, src_r, dst_r, zeros):
    mesh = plsc.VectorSubcoreMesh(
        core_axis_name="c", subcore_axis_name="s", num_cores=NC, num_subcores=NS
    )

    @functools.partial(
        pl.kernel,
        out_type=jax.ShapeDtypeStruct((2 * ACC_ROWS, D), jnp.float32),
        mesh=mesh,
        scratch_types=[
            pltpu.VMEM((KH, C), jnp.int32),
            pltpu.VMEM((KH, C), jnp.int32),
            pltpu.VMEM((C, D), jnp.float32),
            pltpu.VMEM((C, D), jnp.float32),
            pltpu.VMEM_SHARED((ACC_ROWS, D), jnp.float32),
            pltpu.SemaphoreType.DMA,
            pltpu.SemaphoreType.DMA,
        ],
    )
    def segsum(x_hbm, src_hbm, dst_hbm, z_hbm, out_hbm,
               idx_s, idx_d, r0, r1, acc, g0, g1):
        cid = lax.axis_index("c")
        sid = lax.axis_index("s")
        wid = cid * NS + sid

        rs = (r0, r1)
        gs = (g0, g1)

        pltpu.sync_copy(z_hbm, acc.at[pl.ds(sid * RPW, RPW)])

        plsc.subcore_barrier()

        def gather(j, b):
            pltpu.async_copy(x_hbm.at[idx_s.at[j, pl.ds(0, C // 2)]],
                             rs[b].at[pl.ds(0, C // 2)], gs[b])
            pltpu.async_copy(x_hbm.at[idx_s.at[j, pl.ds(C // 2, C // 2)]],
                             rs[b].at[pl.ds(C // 2, C // 2)], gs[b])

        def gather_wait(b):
            pltpu.make_async_copy(x_hbm.at[pl.ds(0, C)], rs[b], gs[b]).wait()

        def scatter_add(j, b):
            pltpu.sync_copy(rs[b], acc.at[idx_d.at[j]], add=True)

        @pl.loop(0, 2)
        def _(h):
            pltpu.sync_copy(src_hbm.at[wid * 2 + h], idx_s)
            pltpu.sync_copy(dst_hbm.at[wid * 2 + h], idx_d)

            gather(0, 0)

            @pl.loop(0, KH // 2)
            def _(jj):
                j0 = 2 * jj
                gather(j0 + 1, 1)
                gather_wait(0)
                scatter_add(j0, 0)

                @pl.when(j0 + 2 < KH)
                def _():
                    gather(j0 + 2, 0)

                gather_wait(1)
                scatter_add(j0 + 1, 1)

        plsc.subcore_barrier()

        pltpu.sync_copy(acc.at[pl.ds(sid * RPW, RPW)],
                        out_hbm.at[pl.ds(cid * ACC_ROWS + sid * RPW, RPW)])

    return segsum(x, src_r, dst_r, zeros)


def _mlp_body(x_ref, p0_ref, p1_ref, wa_ref, ba_ref, wb_ref, bb_ref, o_ref):
    pre = 1.1 * x_ref[...] + p0_ref[0] + p1_ref[0]
    t = _leaky(
        jnp.dot(pre, wa_ref[...], preferred_element_type=jnp.float32,
                precision=lax.Precision.HIGHEST) + ba_ref[...]
    )
    v = jnp.dot(t, wb_ref[...], preferred_element_type=jnp.float32,
                precision=lax.Precision.HIGHEST) + bb_ref[...]
    o_ref[...] = _leaky(_leaky(v))


def _mlp_layer(x, p, wa, ba, wb, bb):
    row = lambda i: (i, 0)
    full = lambda i: (0, 0)
    return pl.pallas_call(
        _mlp_body,
        out_shape=jax.ShapeDtypeStruct((N, D), jnp.float32),
        grid=(GRID,),
        in_specs=[
            pl.BlockSpec((BM, D), row),
            pl.BlockSpec((1, BM, D), lambda i: (0, i, 0)),
            pl.BlockSpec((1, BM, D), lambda i: (1, i, 0)),
            pl.BlockSpec((D, D), full),
            pl.BlockSpec((1, D), full),
            pl.BlockSpec((D, D), full),
            pl.BlockSpec((1, D), full),
        ],
        out_specs=pl.BlockSpec((BM, D), row),
    )(x, p, p, wa, ba, wb, bb)


def _mlp2_body(x_ref, p0_ref, p1_ref, wa_ref, ba_ref, wb_ref, bb_ref,
               wc1_ref, bc1_ref, wc2_ref, bc2_ref, o_ref, acc_ref):
    i = pl.program_id(0)
    pre = 1.1 * x_ref[...] + p0_ref[0] + p1_ref[0]
    t = _leaky(
        jnp.dot(pre, wa_ref[...], preferred_element_type=jnp.float32,
                precision=lax.Precision.HIGHEST) + ba_ref[...]
    )
    v = jnp.dot(t, wb_ref[...], preferred_element_type=jnp.float32,
                precision=lax.Precision.HIGHEST) + bb_ref[...]
    h2 = _leaky(_leaky(v))
    s = jnp.sum(h2, axis=0, keepdims=True)

    @pl.when(i == 0)
    def _():
        acc_ref[...] = s

    @pl.when(i > 0)
    def _():
        acc_ref[...] += s

    @pl.when(i == pl.num_programs(0) - 1)
    def _():
        em = acc_ref[...]
        z = _leaky(
            jnp.dot(em, wc1_ref[...], preferred_element_type=jnp.float32,
                    precision=lax.Precision.HIGHEST) + bc1_ref[...]
        )
        o_ref[...] = jnp.dot(
            z, wc2_ref[...], preferred_element_type=jnp.float32,
            precision=lax.Precision.HIGHEST) + bc2_ref[...]


def _mlp_layer2(h, p, wa, ba, wb, bb, wc1, bc1, wc2p, bc2p):
    row = lambda i: (i, 0)
    full = lambda i: (0, 0)
    return pl.pallas_call(
        _mlp2_body,
        out_shape=jax.ShapeDtypeStruct((1, D), jnp.float32),
        grid=(GRID,),
        in_specs=[
            pl.BlockSpec((BM, D), row),
            pl.BlockSpec((1, BM, D), lambda i: (0, i, 0)),
            pl.BlockSpec((1, BM, D), lambda i: (1, i, 0)),
            pl.BlockSpec((D, D), full),
            pl.BlockSpec((1, D), full),
            pl.BlockSpec((D, D), full),
            pl.BlockSpec((1, D), full),
            pl.BlockSpec((D, D), full),
            pl.BlockSpec((1, D), full),
            pl.BlockSpec((D, D), full),
            pl.BlockSpec((1, D), full),
        ],
        out_specs=pl.BlockSpec((1, D), full),
        scratch_shapes=[pltpu.VMEM((1, D), jnp.float32)],
    )(h, p, p, wa, ba, wb, bb, wc1, bc1, wc2p, bc2p)


def kernel(x, edge_index, W0a, b0a, W0b, b0b, W1a, b1a, W1b, b1b,
           Wc1, bc1, Wc2, bc2):
    src = edge_index[0].astype(jnp.int32).reshape(NW, EW)
    dst = edge_index[1].astype(jnp.int32).reshape(NW, EW)
    src_r = jnp.pad(src, ((0, 0), (0, EW_PAD - EW))).reshape(NW * 2, KH, C)
    dst_r = jnp.pad(dst, ((0, 0), (0, EW_PAD - EW)),
                    constant_values=TRASH).reshape(NW * 2, KH, C)
    zeros = jnp.zeros((RPW, D), jnp.float32)

    b0a_, b0b_, b1a_, b1b_, bc1_ = (
        b.reshape(1, D) for b in (b0a, b0b, b1a, b1b, bc1))
    Wc2p = jnp.pad(Wc2, ((0, 0), (0, D - Wc2.shape[1])))
    bc2p = jnp.pad(bc2, (0, D - bc2.shape[0])).reshape(1, D)

    p1 = _segment_sum_sc(x, src_r, dst_r, zeros).reshape(2, ACC_ROWS, D)
    h1 = _mlp_layer(x, p1, W0a, b0a_, W0b, b0b_)
    p2 = _segment_sum_sc(h1, src_r, dst_r, zeros).reshape(2, ACC_ROWS, D)
    cl = _mlp_layer2(h1, p2, W1a, b1a_, W1b, b1b_, Wc1, bc1_, Wc2p, bc2p)
    return cl[:, : Wc2.shape[1]]


# speedup vs baseline: 1.0047x; 1.0047x over previous
import functools

import jax
import jax.numpy as jnp
from jax import lax
from jax.experimental import pallas as pl
from jax.experimental.pallas import tpu as pltpu
from jax.experimental.pallas import tpu_sc as plsc

N = 10000
D = 128
E = 320000
NC = 2
NS = 16
NW = NC * NS
EW = E // NW
C = 128
K = 80
KH = K // 2
EW_PAD = K * C
RPW = 632
ACC_ROWS = NS * RPW
TRASH = N
BM = 1000
GRID = N // BM


def _leaky(v):
    return jnp.where(v > 0, v, 0.01 * v)


def _segment_sum_sc(x, src_r, dst_r, zeros):
    mesh = plsc.VectorSubcoreMesh(
        core_axis_name="c", subcore_axis_name="s", num_cores=NC, num_subcores=NS
    )

    @functools.partial(
        pl.kernel,
        out_type=jax.ShapeDtypeStruct((2 * ACC_ROWS, D), jnp.float32),
        mesh=mesh,
        scratch_types=[
            pltpu.VMEM((KH, C), jnp.int32),
            pltpu.VMEM((KH, C), jnp.int32),
            pltpu.VMEM((C, D), jnp.float32),
            pltpu.VMEM((C, D), jnp.float32),
            pltpu.VMEM_SHARED((ACC_ROWS, D), jnp.float32),
            pltpu.SemaphoreType.DMA,
            pltpu.SemaphoreType.DMA,
            pltpu.SemaphoreType.DMA,
        ],
    )
    def segsum(x_hbm, src_hbm, dst_hbm, z_hbm, out_hbm,
               idx_s, idx_d, r0, r1, acc, g0, g1, zs):
        cid = lax.axis_index("c")
        sid = lax.axis_index("s")
        wid = cid * NS + sid

        rs = (r0, r1)
        gs = (g0, g1)

        def gather(j, b):
            pltpu.async_copy(x_hbm.at[idx_s.at[j]], rs[b], gs[b])

        def gather_wait(b):
            pltpu.make_async_copy(x_hbm.at[pl.ds(0, C)], rs[b], gs[b]).wait()

        def scatter_add(j, b):
            pltpu.sync_copy(rs[b], acc.at[idx_d.at[j]], add=True)

        pltpu.async_copy(z_hbm, acc.at[pl.ds(sid * RPW, RPW)], zs)
        pltpu.sync_copy(src_hbm.at[wid * 2], idx_s)
        pltpu.sync_copy(dst_hbm.at[wid * 2], idx_d)
        gather(0, 0)
        pltpu.make_async_copy(z_hbm, acc.at[pl.ds(sid * RPW, RPW)], zs).wait()

        plsc.subcore_barrier()

        @pl.loop(0, 2)
        def _(h):
            @pl.when(h > 0)
            def _():
                pltpu.sync_copy(src_hbm.at[wid * 2 + h], idx_s)
                pltpu.sync_copy(dst_hbm.at[wid * 2 + h], idx_d)
                gather(0, 0)

            @pl.loop(0, KH // 2)
            def _(jj):
                j0 = 2 * jj
                gather(j0 + 1, 1)
                gather_wait(0)
                scatter_add(j0, 0)

                @pl.when(j0 + 2 < KH)
                def _():
                    gather(j0 + 2, 0)

                gather_wait(1)
                scatter_add(j0 + 1, 1)

        plsc.subcore_barrier()

        pltpu.sync_copy(acc.at[pl.ds(sid * RPW, RPW)],
                        out_hbm.at[pl.ds(cid * ACC_ROWS + sid * RPW, RPW)])

    return segsum(x, src_r, dst_r, zeros)


def _mlp_body(x_ref, p0_ref, p1_ref, wa_ref, ba_ref, wb_ref, bb_ref, o_ref):
    pre = 1.1 * x_ref[...] + p0_ref[0] + p1_ref[0]
    t = _leaky(
        jnp.dot(pre, wa_ref[...], preferred_element_type=jnp.float32,
                precision=lax.Precision.HIGHEST) + ba_ref[...]
    )
    v = jnp.dot(t, wb_ref[...], preferred_element_type=jnp.float32,
                precision=lax.Precision.HIGHEST) + bb_ref[...]
    o_ref[...] = _leaky(_leaky(v))


def _mlp_layer(x, p, wa, ba, wb, bb):
    row = lambda i: (i, 0)
    full = lambda i: (0, 0)
    return pl.pallas_call(
        _mlp_body,
        out_shape=jax.ShapeDtypeStruct((N, D), jnp.float32),
        grid=(GRID,),
        in_specs=[
            pl.BlockSpec((BM, D), row),
            pl.BlockSpec((1, BM, D), lambda i: (0, i, 0)),
            pl.BlockSpec((1, BM, D), lambda i: (1, i, 0)),
            pl.BlockSpec((D, D), full),
            pl.BlockSpec((1, D), full),
            pl.BlockSpec((D, D), full),
            pl.BlockSpec((1, D), full),
        ],
        out_specs=pl.BlockSpec((BM, D), row),
    )(x, p, p, wa, ba, wb, bb)


def _mlp2_body(x_ref, p0_ref, p1_ref, wa_ref, ba_ref, wb_ref, bb_ref,
               wc1_ref, bc1_ref, wc2_ref, bc2_ref, o_ref, acc_ref):
    i = pl.program_id(0)
    pre = 1.1 * x_ref[...] + p0_ref[0] + p1_ref[0]
    t = _leaky(
        jnp.dot(pre, wa_ref[...], preferred_element_type=jnp.float32,
                precision=lax.Precision.HIGHEST) + ba_ref[...]
    )
    v = jnp.dot(t, wb_ref[...], preferred_element_type=jnp.float32,
                precision=lax.Precision.HIGHEST) + bb_ref[...]
    h2 = _leaky(_leaky(v))
    s = jnp.sum(h2, axis=0, keepdims=True)

    @pl.when(i == 0)
    def _():
        acc_ref[...] = s

    @pl.when(i > 0)
    def _():
        acc_ref[...] += s

    @pl.when(i == pl.num_programs(0) - 1)
    def _():
        em = acc_ref[...]
        z = _leaky(
            jnp.dot(em, wc1_ref[...], preferred_element_type=jnp.float32,
                    precision=lax.Precision.HIGHEST) + bc1_ref[...]
        )
        o_ref[...] = jnp.dot(
            z, wc2_ref[...], preferred_element_type=jnp.float32,
            precision=lax.Precision.HIGHEST) + bc2_ref[...]


def _mlp_layer2(h, p, wa, ba, wb, bb, wc1, bc1, wc2p, bc2p):
    row = lambda i: (i, 0)
    full = lambda i: (0, 0)
    return pl.pallas_call(
        _mlp2_body,
        out_shape=jax.ShapeDtypeStruct((1, D), jnp.float32),
        grid=(GRID,),
        in_specs=[
            pl.BlockSpec((BM, D), row),
            pl.BlockSpec((1, BM, D), lambda i: (0, i, 0)),
            pl.BlockSpec((1, BM, D), lambda i: (1, i, 0)),
            pl.BlockSpec((D, D), full),
            pl.BlockSpec((1, D), full),
            pl.BlockSpec((D, D), full),
            pl.BlockSpec((1, D), full),
            pl.BlockSpec((D, D), full),
            pl.BlockSpec((1, D), full),
            pl.BlockSpec((D, D), full),
            pl.BlockSpec((1, D), full),
        ],
        out_specs=pl.BlockSpec((1, D), full),
        scratch_shapes=[pltpu.VMEM((1, D), jnp.float32)],
    )(h, p, p, wa, ba, wb, bb, wc1, bc1, wc2p, bc2p)


def kernel(x, edge_index, W0a, b0a, W0b, b0b, W1a, b1a, W1b, b1b,
           Wc1, bc1, Wc2, bc2):
    src = edge_index[0].astype(jnp.int32).reshape(NW, EW)
    dst = edge_index[1].astype(jnp.int32).reshape(NW, EW)
    src_r = jnp.pad(src, ((0, 0), (0, EW_PAD - EW))).reshape(NW * 2, KH, C)
    dst_r = jnp.pad(dst, ((0, 0), (0, EW_PAD - EW)),
                    constant_values=TRASH).reshape(NW * 2, KH, C)
    zeros = jnp.zeros((RPW, D), jnp.float32)

    b0a_, b0b_, b1a_, b1b_, bc1_ = (
        b.reshape(1, D) for b in (b0a, b0b, b1a, b1b, bc1))
    Wc2p = jnp.pad(Wc2, ((0, 0), (0, D - Wc2.shape[1])))
    bc2p = jnp.pad(bc2, (0, D - bc2.shape[0])).reshape(1, D)

    p1 = _segment_sum_sc(x, src_r, dst_r, zeros).reshape(2, ACC_ROWS, D)
    h1 = _mlp_layer(x, p1, W0a, b0a_, W0b, b0b_)
    p2 = _segment_sum_sc(h1, src_r, dst_r, zeros).reshape(2, ACC_ROWS, D)
    cl = _mlp_layer2(h1, p2, W1a, b1a_, W1b, b1b_, Wc1, bc1_, Wc2p, bc2p)
    return cl[:, : Wc2.shape[1]]


# speedup vs baseline: 1.0407x; 1.0358x over previous
import functools

import jax
import jax.numpy as jnp
from jax import lax
from jax.experimental import pallas as pl
from jax.experimental.pallas import tpu as pltpu
from jax.experimental.pallas import tpu_sc as plsc

N = 10000
D = 128
E = 320000
NC = 2
NS = 16
NW = NC * NS
EW = E // NW
C = 128
K = 80
KH = K // 2
EW_PAD = K * C
RPW = 632
ACC_ROWS = NS * RPW
TRASH = N
BM = 2000
GRID = N // BM


def _leaky(v):
    return jnp.where(v > 0, v, 0.01 * v)


def _segment_sum_sc(x, src_r, dst_r, zeros):
    mesh = plsc.VectorSubcoreMesh(
        core_axis_name="c", subcore_axis_name="s", num_cores=NC, num_subcores=NS
    )

    @functools.partial(
        pl.kernel,
        out_type=jax.ShapeDtypeStruct((2 * ACC_ROWS, D), jnp.float32),
        mesh=mesh,
        scratch_types=[
            pltpu.VMEM((KH, C), jnp.int32),
            pltpu.VMEM((KH, C), jnp.int32),
            pltpu.VMEM((C, D), jnp.float32),
            pltpu.VMEM((C, D), jnp.float32),
            pltpu.VMEM_SHARED((ACC_ROWS, D), jnp.float32),
            pltpu.SemaphoreType.DMA,
            pltpu.SemaphoreType.DMA,
            pltpu.SemaphoreType.DMA,
        ],
    )
    def segsum(x_hbm, src_hbm, dst_hbm, z_hbm, out_hbm,
               idx_s, idx_d, r0, r1, acc, g0, g1, zs):
        cid = lax.axis_index("c")
        sid = lax.axis_index("s")
        wid = cid * NS + sid

        rs = (r0, r1)
        gs = (g0, g1)

        def gather(j, b):
            pltpu.async_copy(x_hbm.at[idx_s.at[j]], rs[b], gs[b])

        def gather_wait(b):
            pltpu.make_async_copy(x_hbm.at[pl.ds(0, C)], rs[b], gs[b]).wait()

        def scatter_add(j, b):
            pltpu.sync_copy(rs[b], acc.at[idx_d.at[j]], add=True)

        pltpu.async_copy(z_hbm, acc.at[pl.ds(sid * RPW, RPW)], zs)
        pltpu.sync_copy(src_hbm.at[wid * 2], idx_s)
        pltpu.sync_copy(dst_hbm.at[wid * 2], idx_d)
        gather(0, 0)
        pltpu.make_async_copy(z_hbm, acc.at[pl.ds(sid * RPW, RPW)], zs).wait()

        plsc.subcore_barrier()

        @pl.loop(0, 2)
        def _(h):
            @pl.when(h > 0)
            def _():
                pltpu.sync_copy(src_hbm.at[wid * 2 + h], idx_s)
                pltpu.sync_copy(dst_hbm.at[wid * 2 + h], idx_d)
                gather(0, 0)

            @pl.loop(0, KH // 2)
            def _(jj):
                j0 = 2 * jj
                gather(j0 + 1, 1)
                gather_wait(0)
                scatter_add(j0, 0)

                @pl.when(j0 + 2 < KH)
                def _():
                    gather(j0 + 2, 0)

                gather_wait(1)
                scatter_add(j0 + 1, 1)

        plsc.subcore_barrier()

        pltpu.sync_copy(acc.at[pl.ds(sid * RPW, RPW)],
                        out_hbm.at[pl.ds(cid * ACC_ROWS + sid * RPW, RPW)])

    return segsum(x, src_r, dst_r, zeros)


def _mlp_body(x_ref, p0_ref, p1_ref, wa_ref, ba_ref, wb_ref, bb_ref, o_ref):
    pre = 1.1 * x_ref[...] + p0_ref[0] + p1_ref[0]
    t = _leaky(
        jnp.dot(pre, wa_ref[...], preferred_element_type=jnp.float32,
                precision=lax.Precision.HIGHEST) + ba_ref[...]
    )
    v = jnp.dot(t, wb_ref[...], preferred_element_type=jnp.float32,
                precision=lax.Precision.HIGHEST) + bb_ref[...]
    o_ref[...] = _leaky(_leaky(v))


def _mlp_layer(x, p, wa, ba, wb, bb):
    row = lambda i: (i, 0)
    full = lambda i: (0, 0)
    return pl.pallas_call(
        _mlp_body,
        out_shape=jax.ShapeDtypeStruct((N, D), jnp.float32),
        grid=(GRID,),
        in_specs=[
            pl.BlockSpec((BM, D), row),
            pl.BlockSpec((1, BM, D), lambda i: (0, i, 0)),
            pl.BlockSpec((1, BM, D), lambda i: (1, i, 0)),
            pl.BlockSpec((D, D), full),
            pl.BlockSpec((1, D), full),
            pl.BlockSpec((D, D), full),
            pl.BlockSpec((1, D), full),
        ],
        out_specs=pl.BlockSpec((BM, D), row),
    )(x, p, p, wa, ba, wb, bb)


def _mlp2_body(x_ref, p0_ref, p1_ref, wa_ref, ba_ref, wb_ref, bb_ref,
               wc1_ref, bc1_ref, wc2_ref, bc2_ref, o_ref, acc_ref):
    i = pl.program_id(0)
    pre = 1.1 * x_ref[...] + p0_ref[0] + p1_ref[0]
    t = _leaky(
        jnp.dot(pre, wa_ref[...], preferred_element_type=jnp.float32,
                precision=lax.Precision.HIGHEST) + ba_ref[...]
    )
    v = jnp.dot(t, wb_ref[...], preferred_element_type=jnp.float32,
                precision=lax.Precision.HIGHEST) + bb_ref[...]
    h2 = _leaky(_leaky(v))
    s = jnp.sum(h2, axis=0, keepdims=True)

    @pl.when(i == 0)
    def _():
        acc_ref[...] = s

    @pl.when(i > 0)
    def _():
        acc_ref[...] += s

    @pl.when(i == pl.num_programs(0) - 1)
    def _():
        em = acc_ref[...]
        z = _leaky(
            jnp.dot(em, wc1_ref[...], preferred_element_type=jnp.float32,
                    precision=lax.Precision.HIGHEST) + bc1_ref[...]
        )
        o_ref[...] = jnp.dot(
            z, wc2_ref[...], preferred_element_type=jnp.float32,
            precision=lax.Precision.HIGHEST) + bc2_ref[...]


def _mlp_layer2(h, p, wa, ba, wb, bb, wc1, bc1, wc2p, bc2p):
    row = lambda i: (i, 0)
    full = lambda i: (0, 0)
    return pl.pallas_call(
        _mlp2_body,
        out_shape=jax.ShapeDtypeStruct((1, D), jnp.float32),
        grid=(GRID,),
        in_specs=[
            pl.BlockSpec((BM, D), row),
            pl.BlockSpec((1, BM, D), lambda i: (0, i, 0)),
            pl.BlockSpec((1, BM, D), lambda i: (1, i, 0)),
            pl.BlockSpec((D, D), full),
            pl.BlockSpec((1, D), full),
            pl.BlockSpec((D, D), full),
            pl.BlockSpec((1, D), full),
            pl.BlockSpec((D, D), full),
            pl.BlockSpec((1, D), full),
            pl.BlockSpec((D, D), full),
            pl.BlockSpec((1, D), full),
        ],
        out_specs=pl.BlockSpec((1, D), full),
        scratch_shapes=[pltpu.VMEM((1, D), jnp.float32)],
    )(h, p, p, wa, ba, wb, bb, wc1, bc1, wc2p, bc2p)


def kernel(x, edge_index, W0a, b0a, W0b, b0b, W1a, b1a, W1b, b1b,
           Wc1, bc1, Wc2, bc2):
    src = edge_index[0].astype(jnp.int32).reshape(NW, EW)
    dst = edge_index[1].astype(jnp.int32).reshape(NW, EW)
    src_r = jnp.pad(src, ((0, 0), (0, EW_PAD - EW))).reshape(NW * 2, KH, C)
    dst_r = jnp.pad(dst, ((0, 0), (0, EW_PAD - EW)),
                    constant_values=TRASH).reshape(NW * 2, KH, C)
    zeros = jnp.zeros((RPW, D), jnp.float32)

    b0a_, b0b_, b1a_, b1b_, bc1_ = (
        b.reshape(1, D) for b in (b0a, b0b, b1a, b1b, bc1))
    Wc2p = jnp.pad(Wc2, ((0, 0), (0, D - Wc2.shape[1])))
    bc2p = jnp.pad(bc2, (0, D - bc2.shape[0])).reshape(1, D)

    p1 = _segment_sum_sc(x, src_r, dst_r, zeros).reshape(2, ACC_ROWS, D)
    h1 = _mlp_layer(x, p1, W0a, b0a_, W0b, b0b_)
    p2 = _segment_sum_sc(h1, src_r, dst_r, zeros).reshape(2, ACC_ROWS, D)
    cl = _mlp_layer2(h1, p2, W1a, b1a_, W1b, b1b_, Wc1, bc1_, Wc2p, bc2p)
    return cl[:, : Wc2.shape[1]]
